# baseline (speedup 1.0000x reference)
import functools
import math

import jax
import jax.numpy as jnp
from jax import lax
from jax.experimental import pallas as pl
from jax.experimental.pallas import tpu as pltpu
from jax.experimental.pallas import tpu_sc as plsc

N = 2000
H = 600
NG = 50
NI = 6
CUTOFF = 10.0
K = 64
E = N * K
HB = 768
HBW = HB // 2
LN2 = math.log(2.0)
SPACING = CUTOFF / (NG - 1)
COEFF = -0.5 / SPACING**2

_pallas_call = pl.pallas_call

T = 40
BE = T * K
GRID = E // BE


def _ssp(x):
    return jnp.maximum(x, 0.0) + jnp.log1p(jnp.exp(-jnp.abs(x))) - LN2


def _dot(a, b):
    return lax.dot_general(a, b, (((1,), (0,)), ((), ())),
                           preferred_element_type=jnp.float32)


def _mm_body(h_ref, w_ref, o_ref):
    o_ref[...] = _dot(h_ref[...], w_ref[...])


def _matmul(h, w):
    return _pallas_call(
        _mm_body,
        out_shape=jax.ShapeDtypeStruct((h.shape[0], w.shape[1]), jnp.float32),
    )(h, w)


def _gather(xl, src):
    info = plsc.get_sparse_core_info()
    nw = info.num_cores * info.num_subcores
    per_w = E // nw
    ch = 80
    niter = per_w // ch
    mesh = plsc.VectorSubcoreMesh(core_axis_name="c", subcore_axis_name="s")

    @functools.partial(
        pl.kernel,
        out_type=jax.ShapeDtypeStruct((E, HBW), jnp.float32),
        mesh=mesh,
        scratch_types=[
            pltpu.VMEM((ch,), jnp.int32),
            pltpu.VMEM((ch,), jnp.int32),
            pltpu.VMEM((ch, HBW), jnp.float32),
            pltpu.VMEM((ch, HBW), jnp.float32),
            pltpu.SemaphoreType.DMA,
            pltpu.SemaphoreType.DMA,
            pltpu.SemaphoreType.DMA,
            pltpu.SemaphoreType.DMA,
        ],
    )
    def k(x_hbm, i_hbm, o_hbm, idx0, idx1, rows0, rows1, sg0, sg1, so0, so1):
        wid = lax.axis_index("s") * info.num_cores + lax.axis_index("c")
        base = wid * per_w
        idx = (idx0, idx1)
        rows = (rows0, rows1)
        sg = (sg0, sg1)
        so = (so0, so1)

        def load_idx(b, j):
            pltpu.sync_copy(i_hbm.at[pl.ds(base + j * ch, ch)], idx[b])

        def start_gather(b):
            pltpu.async_copy(x_hbm.at[idx[b]], rows[b], sg[b])

        def wait_gather(b):
            pltpu.make_async_copy(x_hbm.at[idx[b]], rows[b], sg[b]).wait()

        def start_out(b, j):
            pltpu.async_copy(rows[b], o_hbm.at[pl.ds(base + j * ch, ch)], so[b])

        def wait_out(b, j):
            pltpu.make_async_copy(
                rows[b], o_hbm.at[pl.ds(base + j * ch, ch)], so[b]).wait()

        load_idx(0, 0)
        start_gather(0)
        load_idx(1, 1)
        wait_gather(0)
        start_out(0, 0)
        start_gather(1)

        def step(jj, carry):
            for b in (0, 1):
                j = 2 * jj + 2 + b
                wait_out(b, j - 2)
                load_idx(b, j)
                start_gather(b)
                wait_gather(1 - b)
                start_out(1 - b, j - 1)
            return carry

        lax.fori_loop(0, (niter - 2) // 2, step, 0)

        wait_gather(1)
        start_out(1, niter - 1)
        wait_out(0, niter - 2)
        wait_out(1, niter - 1)

    return k(xl, src)


def _cfconv_body(d_ref, g_ref, w1_ref, b1_ref, w2_ref, b2_ref, o_ref):
    d = d_ref[...]
    off = lax.broadcasted_iota(jnp.int32, (1, NG), 1).astype(jnp.float32) * SPACING
    ea = jnp.exp(COEFF * (d - off) ** 2)
    w = _ssp(_dot(ea.astype(jnp.bfloat16), w1_ref[...]) + b1_ref[...])
    w = _dot(w.astype(jnp.bfloat16), w2_ref[...]) + b2_ref[...]
    vm = (d < CUTOFF).astype(jnp.float32)
    cv = 0.5 * (jnp.cos(d * (math.pi / CUTOFF)) + 1.0) * vm
    u = lax.bitcast_convert_type(g_ref[...], jnp.uint32)
    glo = lax.bitcast_convert_type(u << 16, jnp.float32)
    ghi = lax.bitcast_convert_type(u & jnp.uint32(0xFFFF0000), jnp.float32)
    g = jnp.concatenate([glo, ghi[:, :H - HBW]], axis=1)
    msg = g * (w * cv)
    o_ref[...] = jnp.sum(msg.reshape(T, K, H), axis=1)


def _cfconv(d_e, g, w1, b1, w2, b2):
    return _pallas_call(
        _cfconv_body,
        grid=(GRID,),
        in_specs=[
            pl.BlockSpec((BE, 1), lambda i: (i, 0)),
            pl.BlockSpec((BE, HBW), lambda i: (i, 0)),
            pl.BlockSpec((NG, H), lambda i: (0, 0)),
            pl.BlockSpec((1, H), lambda i: (0, 0)),
            pl.BlockSpec((H, H), lambda i: (0, 0)),
            pl.BlockSpec((1, H), lambda i: (0, 0)),
        ],
        out_specs=pl.BlockSpec((T, H), lambda i: (i, 0)),
        out_shape=jax.ShapeDtypeStruct((N, H), jnp.float32),
    )(d_e, g, w1, b1, w2, b2)


def _update_body(agg_ref, h_ref, l2w_ref, l2b_ref, lw_ref, lb_ref, o_ref):
    t = _ssp(_dot(agg_ref[...], l2w_ref[...]) + l2b_ref[...])
    o_ref[...] = h_ref[...] + _dot(t, lw_ref[...]) + lb_ref[...]


def _update(agg, h, l2w, l2b, lw, lb):
    return _pallas_call(
        _update_body,
        out_shape=jax.ShapeDtypeStruct((N, H), jnp.float32),
    )(agg, h, l2w, l2b, lw, lb)


def _pack(xl):
    x16 = lax.bitcast_convert_type(xl.astype(jnp.bfloat16), jnp.uint16)
    lo = x16[:, :HBW].astype(jnp.uint32)
    hi = jnp.pad(x16[:, HBW:], ((0, 0), (0, HB - H))).astype(jnp.uint32)
    return lax.bitcast_convert_type(lo | (hi << 16), jnp.float32)


RB = 200


def _dist_body(pos_ref, post_ref, b_ref, bt_ref, o_ref):
    i = pl.program_id(0)
    acc = None
    for c in range(3):
        df = pos_ref[:, c:c + 1] - post_ref[c:c + 1, :]
        acc = df * df if acc is None else acc + df * df
    dm = jnp.sqrt(jnp.maximum(acc, 1e-12))
    lane = lax.broadcasted_iota(jnp.int32, (RB, N), 1)
    row = lax.broadcasted_iota(jnp.int32, (RB, 1), 0) + i * RB
    inval = (lane == row) | (bt_ref[...] != b_ref[...])
    o_ref[...] = jnp.where(inval, 1e9, dm)


def _dist(pos, batch):
    return _pallas_call(
        _dist_body,
        grid=(N // RB,),
        in_specs=[
            pl.BlockSpec((RB, 3), lambda i: (i, 0)),
            pl.BlockSpec((3, N), lambda i: (0, 0)),
            pl.BlockSpec((RB, 1), lambda i: (i, 0)),
            pl.BlockSpec((1, N), lambda i: (0, 0)),
        ],
        out_specs=pl.BlockSpec((RB, N), lambda i: (i, 0)),
        out_shape=jax.ShapeDtypeStruct((N, N), jnp.float32),
    )(pos, pos.T, batch.reshape(N, 1), batch.reshape(1, N))


def kernel(z, pos, batch, params):
    dm = _dist(pos, batch)
    negd, idx = lax.top_k(-dm, K)
    d_e = (-negd).reshape(E, 1)
    src = idx.reshape(-1).astype(jnp.int32)

    p = params
    h = p['emb'][z]
    for i in range(NI):
        xl = _matmul(h, p['lin1_w'][i])
        g = _gather(_pack(xl), src)
        agg = _cfconv(d_e, g,
                      p['mlp_w1'][i].astype(jnp.bfloat16), p['mlp_b1'][i].reshape(1, H),
                      p['mlp_w2'][i].astype(jnp.bfloat16), p['mlp_b2'][i].reshape(1, H))
        h = _update(agg, h,
                    p['lin2_w'][i], p['lin2_b'][i].reshape(1, H),
                    p['lin_w'][i], p['lin_b'][i].reshape(1, H))
    return h


# speedup vs baseline: 1.0110x; 1.0110x over previous
import functools
import math

import jax
import jax.numpy as jnp
from jax import lax
from jax.experimental import pallas as pl
from jax.experimental.pallas import tpu as pltpu
from jax.experimental.pallas import tpu_sc as plsc

N = 2000
H = 600
NG = 50
NI = 6
CUTOFF = 10.0
K = 64
E = N * K
HB = 768
HBW = HB // 2
LN2 = math.log(2.0)
SPACING = CUTOFF / (NG - 1)
COEFF = -0.5 / SPACING**2

_pallas_call = pl.pallas_call

T = 40
BE = T * K
GRID = E // BE


def _ssp(x):
    return jnp.maximum(x, 0.0) + jnp.log1p(jnp.exp(-jnp.abs(x))) - LN2


def _dot(a, b):
    return lax.dot_general(a, b, (((1,), (0,)), ((), ())),
                           preferred_element_type=jnp.float32)


NZ = 100


def _pack_tc(xl):
    xb = jnp.concatenate(
        [xl.astype(jnp.bfloat16),
         jnp.zeros((xl.shape[0], HB - H), jnp.bfloat16)], axis=1)
    x16 = lax.bitcast_convert_type(xb, jnp.uint16)
    lo = x16[:, :HBW].astype(jnp.uint32)
    hi = x16[:, HBW:].astype(jnp.uint32)
    return lax.bitcast_convert_type(lo | (hi << 16), jnp.float32)


def _embed_body(z_ref, emb_ref, w1_ref, h_ref, xp_ref):
    oh = (z_ref[...] == lax.broadcasted_iota(jnp.int32, (N, NZ), 1))
    h = _dot(oh.astype(jnp.float32), emb_ref[...])
    h_ref[...] = h
    xp_ref[...] = _pack_tc(_dot(h, w1_ref[...]))


def _embed(z, emb, w1):
    return _pallas_call(
        _embed_body,
        out_shape=(jax.ShapeDtypeStruct((N, H), jnp.float32),
                   jax.ShapeDtypeStruct((N, HBW), jnp.float32)),
    )(z.reshape(N, 1).astype(jnp.int32), emb, w1)


def _gather(xl, src):
    info = plsc.get_sparse_core_info()
    nw = info.num_cores * info.num_subcores
    per_w = E // nw
    ch = 80
    niter = per_w // ch
    mesh = plsc.VectorSubcoreMesh(core_axis_name="c", subcore_axis_name="s")

    @functools.partial(
        pl.kernel,
        out_type=jax.ShapeDtypeStruct((E, HBW), jnp.float32),
        mesh=mesh,
        scratch_types=[
            pltpu.VMEM((ch,), jnp.int32),
            pltpu.VMEM((ch,), jnp.int32),
            pltpu.VMEM((ch, HBW), jnp.float32),
            pltpu.VMEM((ch, HBW), jnp.float32),
            pltpu.SemaphoreType.DMA,
            pltpu.SemaphoreType.DMA,
            pltpu.SemaphoreType.DMA,
            pltpu.SemaphoreType.DMA,
        ],
    )
    def k(x_hbm, i_hbm, o_hbm, idx0, idx1, rows0, rows1, sg0, sg1, so0, so1):
        wid = lax.axis_index("s") * info.num_cores + lax.axis_index("c")
        base = wid * per_w
        idx = (idx0, idx1)
        rows = (rows0, rows1)
        sg = (sg0, sg1)
        so = (so0, so1)

        def load_idx(b, j):
            pltpu.sync_copy(i_hbm.at[pl.ds(base + j * ch, ch)], idx[b])

        def start_gather(b):
            pltpu.async_copy(x_hbm.at[idx[b]], rows[b], sg[b])

        def wait_gather(b):
            pltpu.make_async_copy(x_hbm.at[idx[b]], rows[b], sg[b]).wait()

        def start_out(b, j):
            pltpu.async_copy(rows[b], o_hbm.at[pl.ds(base + j * ch, ch)], so[b])

        def wait_out(b, j):
            pltpu.make_async_copy(
                rows[b], o_hbm.at[pl.ds(base + j * ch, ch)], so[b]).wait()

        load_idx(0, 0)
        start_gather(0)
        load_idx(1, 1)
        wait_gather(0)
        start_out(0, 0)
        start_gather(1)

        def step(jj, carry):
            for b in (0, 1):
                j = 2 * jj + 2 + b
                wait_out(b, j - 2)
                load_idx(b, j)
                start_gather(b)
                wait_gather(1 - b)
                start_out(1 - b, j - 1)
            return carry

        lax.fori_loop(0, (niter - 2) // 2, step, 0)

        wait_gather(1)
        start_out(1, niter - 1)
        wait_out(0, niter - 2)
        wait_out(1, niter - 1)

    return k(xl, src)


def _cfconv_body(d_ref, g_ref, w1_ref, b1_ref, w2_ref, b2_ref, o_ref):
    d = d_ref[...]
    off = lax.broadcasted_iota(jnp.int32, (1, NG), 1).astype(jnp.float32) * SPACING
    ea = jnp.exp(COEFF * (d - off) ** 2)
    w = _ssp(_dot(ea.astype(jnp.bfloat16), w1_ref[...]) + b1_ref[...])
    w = _dot(w.astype(jnp.bfloat16), w2_ref[...]) + b2_ref[...]
    vm = (d < CUTOFF).astype(jnp.float32)
    cv = 0.5 * (jnp.cos(d * (math.pi / CUTOFF)) + 1.0) * vm
    u = lax.bitcast_convert_type(g_ref[...], jnp.uint32)
    glo = lax.bitcast_convert_type(u << 16, jnp.float32)
    ghi = lax.bitcast_convert_type(u & jnp.uint32(0xFFFF0000), jnp.float32)
    g = jnp.concatenate([glo, ghi[:, :H - HBW]], axis=1)
    msg = g * (w * cv)
    o_ref[...] = jnp.sum(msg.reshape(T, K, H), axis=1)


def _cfconv(d_e, g, w1, b1, w2, b2):
    return _pallas_call(
        _cfconv_body,
        grid=(GRID,),
        in_specs=[
            pl.BlockSpec((BE, 1), lambda i: (i, 0)),
            pl.BlockSpec((BE, HBW), lambda i: (i, 0)),
            pl.BlockSpec((NG, H), lambda i: (0, 0)),
            pl.BlockSpec((1, H), lambda i: (0, 0)),
            pl.BlockSpec((H, H), lambda i: (0, 0)),
            pl.BlockSpec((1, H), lambda i: (0, 0)),
        ],
        out_specs=pl.BlockSpec((T, H), lambda i: (i, 0)),
        out_shape=jax.ShapeDtypeStruct((N, H), jnp.float32),
    )(d_e, g, w1, b1, w2, b2)


def _update_body(agg_ref, h_ref, l2w_ref, l2b_ref, lw_ref, lb_ref, o_ref):
    t = _ssp(_dot(agg_ref[...], l2w_ref[...]) + l2b_ref[...])
    o_ref[...] = h_ref[...] + _dot(t, lw_ref[...]) + lb_ref[...]


def _update(agg, h, l2w, l2b, lw, lb):
    return _pallas_call(
        _update_body,
        out_shape=jax.ShapeDtypeStruct((N, H), jnp.float32),
    )(agg, h, l2w, l2b, lw, lb)


def _update_pack_body(agg_ref, h_ref, l2w_ref, l2b_ref, lw_ref, lb_ref,
                      w1n_ref, hn_ref, xp_ref):
    t = _ssp(_dot(agg_ref[...], l2w_ref[...]) + l2b_ref[...])
    hn = h_ref[...] + _dot(t, lw_ref[...]) + lb_ref[...]
    hn_ref[...] = hn
    xp_ref[...] = _pack_tc(_dot(hn, w1n_ref[...]))


def _update_pack(agg, h, l2w, l2b, lw, lb, w1n):
    return _pallas_call(
        _update_pack_body,
        out_shape=(jax.ShapeDtypeStruct((N, H), jnp.float32),
                   jax.ShapeDtypeStruct((N, HBW), jnp.float32)),
    )(agg, h, l2w, l2b, lw, lb, w1n)


RB = 200


def _dist_body(pos_ref, post_ref, b_ref, bt_ref, o_ref):
    i = pl.program_id(0)
    acc = None
    for c in range(3):
        df = pos_ref[:, c:c + 1] - post_ref[c:c + 1, :]
        acc = df * df if acc is None else acc + df * df
    dm = jnp.sqrt(jnp.maximum(acc, 1e-12))
    lane = lax.broadcasted_iota(jnp.int32, (RB, N), 1)
    row = lax.broadcasted_iota(jnp.int32, (RB, 1), 0) + i * RB
    inval = (lane == row) | (bt_ref[...] != b_ref[...])
    o_ref[...] = jnp.where(inval, 1e9, dm)


def _dist(pos, batch):
    return _pallas_call(
        _dist_body,
        grid=(N // RB,),
        in_specs=[
            pl.BlockSpec((RB, 3), lambda i: (i, 0)),
            pl.BlockSpec((3, N), lambda i: (0, 0)),
            pl.BlockSpec((RB, 1), lambda i: (i, 0)),
            pl.BlockSpec((1, N), lambda i: (0, 0)),
        ],
        out_specs=pl.BlockSpec((RB, N), lambda i: (i, 0)),
        out_shape=jax.ShapeDtypeStruct((N, N), jnp.float32),
    )(pos, pos.T, batch.reshape(N, 1), batch.reshape(1, N))


def kernel(z, pos, batch, params):
    dm = _dist(pos, batch)
    negd, idx = lax.top_k(-dm, K)
    d_e = (-negd).reshape(E, 1)
    src = idx.reshape(-1).astype(jnp.int32)

    p = params
    h, xp = _embed(z, p['emb'], p['lin1_w'][0])
    for i in range(NI):
        g = _gather(xp, src)
        agg = _cfconv(d_e, g,
                      p['mlp_w1'][i].astype(jnp.bfloat16), p['mlp_b1'][i].reshape(1, H),
                      p['mlp_w2'][i].astype(jnp.bfloat16), p['mlp_b2'][i].reshape(1, H))
        if i + 1 < NI:
            h, xp = _update_pack(agg, h,
                                 p['lin2_w'][i], p['lin2_b'][i].reshape(1, H),
                                 p['lin_w'][i], p['lin_b'][i].reshape(1, H),
                                 p['lin1_w'][i + 1])
        else:
            h = _update(agg, h,
                        p['lin2_w'][i], p['lin2_b'][i].reshape(1, H),
                        p['lin_w'][i], p['lin_b'][i].reshape(1, H))
    return h
